# baseline (speedup 1.0000x reference)
import functools

import jax
import jax.numpy as jnp
from jax import lax
from jax.experimental import pallas as pl
from jax.experimental.pallas import tpu as pltpu
from jax.experimental.pallas import tpu_sc as plsc

B = 16384
D = 64
LANES = 16
NUM_CORES = 2
NUM_SUBCORES = 16
NW = NUM_CORES * NUM_SUBCORES
BPW = B // NW
CHUNK = 128
NCHUNK = BPW // CHUNK
RSTEP = 8


def _sc_partials(z3, pidx3, h3, centers128):
    mesh = plsc.VectorSubcoreMesh(core_axis_name="c", subcore_axis_name="s")

    @functools.partial(
        pl.kernel,
        out_type=jax.ShapeDtypeStruct((NW, LANES), jnp.float32),
        mesh=mesh,
        scratch_types=[
            pltpu.VMEM((NCHUNK, CHUNK), jnp.int32),
            pltpu.VMEM((2, CHUNK, 2 * D), jnp.float32),
            pltpu.VMEM((BPW // 2, 2 * D), jnp.float32),
            pltpu.VMEM((BPW // RSTEP, 2 * D), jnp.float32),
            pltpu.VMEM((LANES,), jnp.float32),
            pltpu.SemaphoreType.DMA,
            pltpu.SemaphoreType.DMA,
            pltpu.SemaphoreType.DMA,
        ],
    )
    def k(z_hbm, pidx_hbm, h_hbm, centers_hbm, out_hbm,
          pidx_v, c_v, z_v, h_v, acc_v, gsem0, gsem1, zsem):
        wid = lax.axis_index("s") * NUM_CORES + lax.axis_index("c")
        gsems = (gsem0, gsem1)

        pltpu.sync_copy(pidx_hbm.at[wid], pidx_v)

        def gather(j):
            return pltpu.async_copy(
                centers_hbm.at[pidx_v.at[j]], c_v.at[j % 2], gsems[j % 2])

        zcopy = pltpu.async_copy(z_hbm.at[wid], z_v, zsem)
        hcopy = pltpu.async_copy(h_hbm.at[wid], h_v, zsem)
        pending = [gather(0), gather(1)]
        zcopy.wait()
        hcopy.wait()

        ones = jnp.ones((LANES,), jnp.float32)
        zero = jnp.zeros((LANES,), jnp.float32)
        accs = (zero,) * (D // LANES)

        for jc in range(NCHUNK):
            pending[jc % 2].wait()
            buf = jc % 2

            def body(t, accs, jc=jc, buf=buf):
                accs = list(accs)
                hrow = jc * (CHUNK // RSTEP) + t
                for kk in range(RSTEP):
                    h = h_v[hrow, pl.ds(kk * LANES, LANES)]
                    g1 = ones - h
                    crow = t * RSTEP + kk
                    zrow = jc * (CHUNK // 2) + t * (RSTEP // 2) + kk // 2
                    zoff = (kk % 2) * D
                    for j in range(D // LANES):
                        lo = c_v[buf, crow, pl.ds(j * LANES, LANES)]
                        hi = c_v[buf, crow, pl.ds(D + j * LANES, LANES)]
                        c = lo * g1 + hi * h
                        dz = z_v[zrow, pl.ds(zoff + j * LANES, LANES)] - c
                        accs[j] = accs[j] + dz * dz
                return tuple(accs)

            accs = lax.fori_loop(0, CHUNK // RSTEP, body, accs)
            if jc + 2 < NCHUNK:
                pending[jc % 2] = gather(jc + 2)

        acc_v[...] = accs[0] + accs[1] + accs[2] + accs[3]
        pltpu.sync_copy(acc_v, out_hbm.at[wid])

    return k(z3, pidx3, h3, centers128)


def _reduce_partials(partials):
    def body(p_ref, o_ref):
        o_ref[0, 0] = jnp.sum(p_ref[...]) * (1.0 / B)

    out = pl.pallas_call(
        body,
        out_shape=jax.ShapeDtypeStruct((1, 1), jnp.float32),
        out_specs=pl.BlockSpec(memory_space=pltpu.SMEM),
    )(partials)
    return out[0, 0]


def kernel(z, labels, centers):
    labels = labels.astype(jnp.int32)
    pidx3 = (labels >> 1).reshape(NW, NCHUNK, CHUNK)
    h3 = jnp.broadcast_to(
        (labels & 1).astype(jnp.float32)[:, None], (B, LANES)
    ).reshape(NW, BPW // RSTEP, RSTEP * LANES)
    centers128 = centers.reshape(centers.shape[0] // 2, 2 * D)
    z3 = z.reshape(NW, BPW // 2, 2 * D)
    partials = _sc_partials(z3, pidx3, h3, centers128)
    return _reduce_partials(partials)


# speedup vs baseline: 1.7013x; 1.7013x over previous
import functools

import jax
import jax.numpy as jnp
from jax import lax
from jax.experimental import pallas as pl
from jax.experimental.pallas import tpu as pltpu
from jax.experimental.pallas import tpu_sc as plsc

B = 16384
D = 64
LANES = 16
NUM_CORES = 2
NUM_SUBCORES = 16
NW = NUM_CORES * NUM_SUBCORES
BPW = B // NW
GRP = 16
NGRP = BPW // GRP


def _sc_partials(z, lbl2, centers):
    mesh = plsc.VectorSubcoreMesh(core_axis_name="c", subcore_axis_name="s")

    @functools.partial(
        pl.kernel,
        out_type=jax.ShapeDtypeStruct((NW, LANES), jnp.float32),
        mesh=mesh,
        scratch_types=[
            pltpu.VMEM((BPW,), jnp.int32),
            pltpu.VMEM((BPW // 2, 2 * D), jnp.float32),
            pltpu.VMEM((BPW, D), jnp.float32),
            pltpu.VMEM((LANES,), jnp.float32),
            pltpu.SemaphoreType.DMA,
            pltpu.SemaphoreType.DMA,
        ],
    )
    def k(z_hbm, lbl_hbm, centers_hbm, out_hbm,
          lbl_v, c_v, z_v, acc_v, gsem, zsem):
        wid = lax.axis_index("s") * NUM_CORES + lax.axis_index("c")
        base = wid * BPW

        pltpu.sync_copy(lbl_hbm.at[wid], lbl_v)
        zcopy = pltpu.async_copy(z_hbm.at[pl.ds(base, BPW)], z_v, zsem)

        def issue(g, carry):
            lblv = lbl_v[pl.ds(g * GRP, GRP)]
            for r in range(GRP):
                pltpu.async_copy(
                    centers_hbm.at[lblv[r]],
                    c_v.at[g * (GRP // 2) + r // 2, pl.ds((r % 2) * D, D)],
                    gsem,
                )
            return carry

        lax.fori_loop(0, NGRP, issue, 0)
        zcopy.wait()

        def body(g, accs):
            accs = list(accs)
            for r in range(GRP):
                pltpu.make_async_copy(
                    centers_hbm.at[0],
                    c_v.at[g * (GRP // 2) + r // 2, pl.ds((r % 2) * D, D)],
                    gsem).wait()
            for r in range(GRP):
                crow = g * (GRP // 2) + r // 2
                coff = (r % 2) * D
                for j in range(D // LANES):
                    dz = (z_v[g * GRP + r, pl.ds(j * LANES, LANES)]
                          - c_v[crow, pl.ds(coff + j * LANES, LANES)])
                    accs[j] = accs[j] + dz * dz
            return tuple(accs)

        zero = jnp.zeros((LANES,), jnp.float32)
        accs = lax.fori_loop(0, NGRP, body, (zero,) * (D // LANES))
        acc_v[...] = accs[0] + accs[1] + accs[2] + accs[3]
        pltpu.sync_copy(acc_v, out_hbm.at[wid])

    return k(z, lbl2, centers)


def _reduce_partials(partials):
    def body(p_ref, o_ref):
        o_ref[0, 0] = jnp.sum(p_ref[...]) * (1.0 / B)

    out = pl.pallas_call(
        body,
        out_shape=jax.ShapeDtypeStruct((1, 1), jnp.float32),
        out_specs=pl.BlockSpec(memory_space=pltpu.SMEM),
    )(partials)
    return out[0, 0]


def kernel(z, labels, centers):
    lbl2 = labels.astype(jnp.int32).reshape(NW, BPW)
    partials = _sc_partials(z, lbl2, centers)
    return _reduce_partials(partials)


# speedup vs baseline: 2.5625x; 1.5062x over previous
import functools

import jax
import jax.numpy as jnp
from jax import lax
from jax.experimental import pallas as pl
from jax.experimental.pallas import tpu as pltpu
from jax.experimental.pallas import tpu_sc as plsc

B = 16384
D = 64
LANES = 16
SUBL = 8
NUM_CORES = 2
NUM_SUBCORES = 16
NW = NUM_CORES * NUM_SUBCORES
BPW = B // NW
NGRP = BPW // LANES


def _sc_partials(z4, blk2, sub2, centers3):
    mesh = plsc.VectorSubcoreMesh(core_axis_name="c", subcore_axis_name="s")

    @functools.partial(
        pl.kernel,
        out_type=jax.ShapeDtypeStruct((NW, LANES), jnp.float32),
        mesh=mesh,
        scratch_types=[
            pltpu.VMEM((BPW,), jnp.int32),
            pltpu.VMEM((BPW,), jnp.int32),
            pltpu.VMEM((BPW // 2, 2 * D), jnp.float32),
            pltpu.VMEM((BPW // SUBL, SUBL, D), jnp.float32),
            pltpu.VMEM((LANES,), jnp.float32),
            pltpu.SemaphoreType.DMA,
        ],
    )
    def k(z_hbm, blk_hbm, sub_hbm, centers_hbm, out_hbm,
          blk_v, sub_v, c_v, z_v, acc_v, sem):
        wid = lax.axis_index("s") * NUM_CORES + lax.axis_index("c")

        pltpu.sync_copy(blk_hbm.at[wid], blk_v)
        pltpu.sync_copy(sub_hbm.at[wid], sub_v)

        def issue(g, carry):
            blkv = blk_v[pl.ds(g * LANES, LANES)]
            subv = sub_v[pl.ds(g * LANES, LANES)]
            for r in range(LANES):
                pltpu.async_copy(
                    centers_hbm.at[blkv[r], subv[r]],
                    c_v.at[g * (LANES // 2) + r // 2, pl.ds((r % 2) * D, D)],
                    sem,
                )
            return carry

        lax.fori_loop(0, NGRP, issue, 0)
        pltpu.sync_copy(z_hbm.at[wid], z_v)

        def body(g, accs):
            accs = list(accs)
            for r in range(LANES):
                pltpu.make_async_copy(
                    centers_hbm.at[0, 0],
                    c_v.at[g * (LANES // 2) + r // 2, pl.ds((r % 2) * D, D)],
                    sem).wait()
            for r in range(LANES):
                crow = g * (LANES // 2) + r // 2
                ccol = (r % 2) * D
                zblk = g * (LANES // SUBL) + r // SUBL
                zsub = r % SUBL
                for j in range(D // LANES):
                    dz = (z_v[zblk, zsub, pl.ds(j * LANES, LANES)]
                          - c_v[crow, pl.ds(ccol + j * LANES, LANES)])
                    accs[j] = accs[j] + dz * dz
            return tuple(accs)

        zero = jnp.zeros((LANES,), jnp.float32)
        accs = lax.fori_loop(0, NGRP, body, (zero,) * (D // LANES))
        acc_v[...] = accs[0] + accs[1] + accs[2] + accs[3]
        pltpu.sync_copy(acc_v, out_hbm.at[wid])

    return k(z4, blk2, sub2, centers3)


def _reduce_partials(partials):
    def body(p_ref, o_ref):
        o_ref[0, 0] = jnp.sum(p_ref[...]) * (1.0 / B)

    out = pl.pallas_call(
        body,
        out_shape=jax.ShapeDtypeStruct((1, 1), jnp.float32),
        out_specs=pl.BlockSpec(memory_space=pltpu.SMEM),
    )(partials)
    return out[0, 0]


def kernel(z, labels, centers):
    labels = labels.astype(jnp.int32)
    blk2 = (labels >> 3).reshape(NW, BPW)
    sub2 = (labels & 7).reshape(NW, BPW)
    centers3 = centers.reshape(centers.shape[0] // SUBL, SUBL, D)
    z4 = z.reshape(NW, BPW // SUBL, SUBL, D)
    partials = _sc_partials(z4, blk2, sub2, centers3)
    return _reduce_partials(partials)
